# baseline (speedup 1.0000x reference)
import functools

import jax
import jax.numpy as jnp
from jax import lax
from jax.experimental import pallas as pl
from jax.experimental.pallas import tpu as pltpu
from jax.experimental.pallas import tpu_sc as plsc

_B, _C, _M = 4, 3, 4096
_N = 4096
_MB = 1024
_TCB = 3
_MI = _M // _MB
_EPS = 1e-8

_NC = 2
_NS = 16
_NW = _NC * _NS
_ROWS_PER_TILE = _M // _NW
_L = 16
_CHUNKS = _N // _L


def _tc_main(p_ref, g_ref, out_ref, colmin_ref, facc_ref, bacc_ref):
    b = pl.program_id(0)
    mi = pl.program_id(1)

    @pl.when(mi == 0)
    def _init_batch():
        colmin_ref[...] = jnp.full((1, _N), jnp.inf, jnp.float32)

    @pl.when((b == 0) & (mi == 0))
    def _init_all():
        facc_ref[...] = jnp.zeros((1, 1), jnp.float32)
        bacc_ref[...] = jnp.zeros((1, 1), jnp.float32)

    p = p_ref[0]
    g = g_ref[0]
    p0 = p[0, :].reshape(_MB, 1)
    p1 = p[1, :].reshape(_MB, 1)
    p2 = p[2, :].reshape(_MB, 1)
    g0 = g[0, :].reshape(1, _N)
    g1 = g[1, :].reshape(1, _N)
    g2 = g[2, :].reshape(1, _N)
    q0 = -2.0 * p0
    q1 = -2.0 * p1
    q2 = -2.0 * p2
    np2 = p0 * p0 + p1 * p1 + p2 * p2
    ng2 = g0 * g0 + g1 * g1 + g2 * g2
    t = q0 * g0 + q1 * g1 + q2 * g2
    e = t + ng2
    f = t + np2

    rowmin = np2 + jnp.min(e, axis=1, keepdims=True)
    facc_ref[...] += jnp.sum(jnp.sqrt(rowmin + _EPS), axis=(0, 1), keepdims=True)
    colmin_ref[...] = jnp.minimum(colmin_ref[...], jnp.min(f, axis=0, keepdims=True))

    @pl.when(mi == _MI - 1)
    def _finish_batch():
        bmin = ng2 + colmin_ref[...]
        bacc_ref[...] += jnp.sum(jnp.sqrt(bmin + _EPS), axis=(0, 1), keepdims=True)

    @pl.when((b == _TCB - 1) & (mi == _MI - 1))
    def _emit():
        out_ref[...] = jnp.concatenate([facc_ref[...], bacc_ref[...]], axis=1)


def _sc_body(p0_hbm, p1_hbm, p2_hbm, g0_hbm, g1_hbm, g2_hbm, rowph_hbm, colph_hbm, g0v, g1v, g2v, pv0, pv1, pv2, cminv, rowv, sem):
    cid = lax.axis_index("c")
    sid = lax.axis_index("s")
    wid = sid * _NC + cid
    base = wid * _ROWS_PER_TILE

    pltpu.sync_copy(g0_hbm, g0v)
    pltpu.sync_copy(g1_hbm, g1v)
    pltpu.sync_copy(g2_hbm, g2v)
    pltpu.sync_copy(p0_hbm.at[pl.ds(base * _L, _ROWS_PER_TILE * _L)], pv0)
    pltpu.sync_copy(p1_hbm.at[pl.ds(base * _L, _ROWS_PER_TILE * _L)], pv1)
    pltpu.sync_copy(p2_hbm.at[pl.ds(base * _L, _ROWS_PER_TILE * _L)], pv2)

    big = jnp.full((_L,), 3.0e38, jnp.float32)
    for j in range(_CHUNKS):
        cminv[pl.ds(j * _L, _L)] = big

    def row_body(m, _):
        o_m = m * _L
        p0s = pv0[pl.ds(o_m, _L)]
        p1s = pv1[pl.ds(o_m, _L)]
        p2s = pv2[pl.ds(o_m, _L)]

        def chunk_body(j, racc):
            o = j * _L
            d0 = p0s - g0v[pl.ds(o, _L)]
            d1 = p1s - g1v[pl.ds(o, _L)]
            d2 = p2s - g2v[pl.ds(o, _L)]
            d = d0 * d0 + d1 * d1 + d2 * d2
            c = cminv[pl.ds(o, _L)]
            cminv[pl.ds(o, _L)] = jnp.minimum(c, d)
            return jnp.minimum(racc, d)

        racc = lax.fori_loop(0, _CHUNKS, chunk_body, big)
        rowv[pl.ds(m * _L, _L)] = racc
        return 0

    lax.fori_loop(0, _ROWS_PER_TILE, row_body, 0)

    pltpu.sync_copy(rowv, rowph_hbm.at[pl.ds(base * _L, _ROWS_PER_TILE * _L)])
    pltpu.sync_copy(cminv, colph_hbm.at[pl.ds(wid * _N, _N)])


@functools.partial(
    pl.kernel,
    out_type=(
        jax.ShapeDtypeStruct((_M * _L,), jnp.float32),
        jax.ShapeDtypeStruct((_NW * _N,), jnp.float32),
    ),
    mesh=plsc.VectorSubcoreMesh(core_axis_name="c", subcore_axis_name="s"),
    scratch_types=[
        pltpu.VMEM((_N,), jnp.float32),
        pltpu.VMEM((_N,), jnp.float32),
        pltpu.VMEM((_N,), jnp.float32),
        pltpu.VMEM((_ROWS_PER_TILE * _L,), jnp.float32),
        pltpu.VMEM((_ROWS_PER_TILE * _L,), jnp.float32),
        pltpu.VMEM((_ROWS_PER_TILE * _L,), jnp.float32),
        pltpu.VMEM((_N,), jnp.float32),
        pltpu.VMEM((_ROWS_PER_TILE * _L,), jnp.float32),
        pltpu.SemaphoreType.DMA,
    ],
)
def _sc_minima(p0_hbm, p1_hbm, p2_hbm, g0_hbm, g1_hbm, g2_hbm, rowph_hbm, colph_hbm, g0v, g1v, g2v, pv0, pv1, pv2, cminv, rowv, sem):
    _sc_body(p0_hbm, p1_hbm, p2_hbm, g0_hbm, g1_hbm, g2_hbm, rowph_hbm, colph_hbm, g0v, g1v, g2v, pv0, pv1, pv2, cminv, rowv, sem)


def _tc_merge(tcpart_ref, rowph_ref, colph_ref, out_ref):
    rmin = jnp.min(rowph_ref[...], axis=1, keepdims=True)
    fsum = jnp.sum(jnp.sqrt(rmin + _EPS), axis=(0, 1), keepdims=True)
    cmin = jnp.min(colph_ref[...], axis=0, keepdims=True)
    bsum = jnp.sum(jnp.sqrt(cmin + _EPS), axis=(0, 1), keepdims=True)
    ftot = tcpart_ref[:, 0:1] + fsum
    btot = tcpart_ref[:, 1:2] + bsum
    out_ref[...] = ftot / (_B * _M) + btot / (_B * _N)


def kernel(predict_pc, gt_pc):
    p_tc = predict_pc[:_TCB]
    g_tc = gt_pc[:_TCB]
    tc_part = pl.pallas_call(
        _tc_main,
        grid=(_TCB, _MI),
        in_specs=[
            pl.BlockSpec((1, _C, _MB), lambda b, mi: (b, 0, mi)),
            pl.BlockSpec((1, _C, _N), lambda b, mi: (b, 0, 0)),
        ],
        out_specs=pl.BlockSpec((1, 2), lambda b, mi: (0, 0)),
        out_shape=jax.ShapeDtypeStruct((1, 2), jnp.float32),
        scratch_shapes=[
            pltpu.VMEM((1, _N), jnp.float32),
            pltpu.VMEM((1, 1), jnp.float32),
            pltpu.VMEM((1, 1), jnp.float32),
        ],
    )(p_tc, g_tc)

    p_sc = predict_pc[_TCB]
    g_sc = gt_pc[_TCB]
    p_b = jnp.broadcast_to(p_sc[:, :, None], (_C, _M, _L)).reshape(_C, _M * _L)
    rowph, colph = _sc_minima(
        p_b[0], p_b[1], p_b[2], g_sc[0], g_sc[1], g_sc[2]
    )

    out = pl.pallas_call(
        _tc_merge,
        out_shape=jax.ShapeDtypeStruct((1, 1), jnp.float32),
    )(tc_part, rowph.reshape(_M, _L), colph.reshape(_NW, _N))
    return out[0, 0]


# speedup vs baseline: 1.1922x; 1.1922x over previous
import functools

import jax
import jax.numpy as jnp
from jax import lax
from jax.experimental import pallas as pl
from jax.experimental.pallas import tpu as pltpu
from jax.experimental.pallas import tpu_sc as plsc

_B, _C, _M = 4, 3, 4096
_N = 4096
_MB = 1024
_TCB = 3
_MI = _M // _MB
_EPS = 1e-8

_NC = 2
_NS = 16
_NW = _NC * _NS
_ROWS_PER_TILE = _M // _NW
_L = 16
_CHUNKS = _N // _L
_R = 4


def _tc_main(p_ref, g_ref, out_ref, colmin_ref, facc_ref, bacc_ref):
    b = pl.program_id(0)
    mi = pl.program_id(1)

    @pl.when(mi == 0)
    def _init_batch():
        colmin_ref[...] = jnp.full((1, _N), jnp.inf, jnp.float32)

    @pl.when((b == 0) & (mi == 0))
    def _init_all():
        facc_ref[...] = jnp.zeros((1, 1), jnp.float32)
        bacc_ref[...] = jnp.zeros((1, 1), jnp.float32)

    p = p_ref[0]
    g = g_ref[0]
    p0 = p[0, :].reshape(_MB, 1)
    p1 = p[1, :].reshape(_MB, 1)
    p2 = p[2, :].reshape(_MB, 1)
    g0 = g[0, :].reshape(1, _N)
    g1 = g[1, :].reshape(1, _N)
    g2 = g[2, :].reshape(1, _N)
    q0 = -2.0 * p0
    q1 = -2.0 * p1
    q2 = -2.0 * p2
    np2 = p0 * p0 + p1 * p1 + p2 * p2
    ng2 = g0 * g0 + g1 * g1 + g2 * g2
    t = q0 * g0 + q1 * g1 + q2 * g2
    e = t + ng2
    f = t + np2

    rowmin = np2 + jnp.min(e, axis=1, keepdims=True)
    facc_ref[...] += jnp.sum(jnp.sqrt(rowmin + _EPS), axis=(0, 1), keepdims=True)
    colmin_ref[...] = jnp.minimum(colmin_ref[...], jnp.min(f, axis=0, keepdims=True))

    @pl.when(mi == _MI - 1)
    def _finish_batch():
        bmin = ng2 + colmin_ref[...]
        bacc_ref[...] += jnp.sum(jnp.sqrt(bmin + _EPS), axis=(0, 1), keepdims=True)

    @pl.when((b == _TCB - 1) & (mi == _MI - 1))
    def _emit():
        out_ref[...] = jnp.concatenate([facc_ref[...], bacc_ref[...]], axis=1)


def _sc_body(p0_hbm, p1_hbm, p2_hbm, g0_hbm, g1_hbm, g2_hbm, rowph_hbm, colph_hbm, g0v, g1v, g2v, pv0, pv1, pv2, cminv, rowv, sem):
    cid = lax.axis_index("c")
    sid = lax.axis_index("s")
    wid = sid * _NC + cid
    base = wid * _ROWS_PER_TILE

    pltpu.sync_copy(g0_hbm, g0v)
    pltpu.sync_copy(g1_hbm, g1v)
    pltpu.sync_copy(g2_hbm, g2v)
    pltpu.sync_copy(p0_hbm.at[pl.ds(base * _L, _ROWS_PER_TILE * _L)], pv0)
    pltpu.sync_copy(p1_hbm.at[pl.ds(base * _L, _ROWS_PER_TILE * _L)], pv1)
    pltpu.sync_copy(p2_hbm.at[pl.ds(base * _L, _ROWS_PER_TILE * _L)], pv2)

    big = jnp.full((_L,), 3.0e38, jnp.float32)
    for j in range(_CHUNKS):
        cminv[pl.ds(j * _L, _L)] = big

    def block_body(mb, _):
        ob = mb * (_R * _L)
        ps = [
            (pv0[pl.ds(ob + r * _L, _L)],
             pv1[pl.ds(ob + r * _L, _L)],
             pv2[pl.ds(ob + r * _L, _L)])
            for r in range(_R)
        ]

        def chunk_body(j, raccs):
            o = j * _L
            g0c = g0v[pl.ds(o, _L)]
            g1c = g1v[pl.ds(o, _L)]
            g2c = g2v[pl.ds(o, _L)]
            ds_list = []
            for r in range(_R):
                d0 = ps[r][0] - g0c
                d1 = ps[r][1] - g1c
                d2 = ps[r][2] - g2c
                ds_list.append(d0 * d0 + d1 * d1 + d2 * d2)
            c = cminv[pl.ds(o, _L)]
            for r in range(_R):
                c = jnp.minimum(c, ds_list[r])
            cminv[pl.ds(o, _L)] = c
            return tuple(jnp.minimum(raccs[r], ds_list[r]) for r in range(_R))

        raccs = lax.fori_loop(0, _CHUNKS, chunk_body, (big,) * _R, unroll=2)
        for r in range(_R):
            rowv[pl.ds(ob + r * _L, _L)] = raccs[r]
        return 0

    lax.fori_loop(0, _ROWS_PER_TILE // _R, block_body, 0)

    pltpu.sync_copy(rowv, rowph_hbm.at[pl.ds(base * _L, _ROWS_PER_TILE * _L)])
    pltpu.sync_copy(cminv, colph_hbm.at[pl.ds(wid * _N, _N)])


@functools.partial(
    pl.kernel,
    out_type=(
        jax.ShapeDtypeStruct((_M * _L,), jnp.float32),
        jax.ShapeDtypeStruct((_NW * _N,), jnp.float32),
    ),
    mesh=plsc.VectorSubcoreMesh(core_axis_name="c", subcore_axis_name="s"),
    scratch_types=[
        pltpu.VMEM((_N,), jnp.float32),
        pltpu.VMEM((_N,), jnp.float32),
        pltpu.VMEM((_N,), jnp.float32),
        pltpu.VMEM((_ROWS_PER_TILE * _L,), jnp.float32),
        pltpu.VMEM((_ROWS_PER_TILE * _L,), jnp.float32),
        pltpu.VMEM((_ROWS_PER_TILE * _L,), jnp.float32),
        pltpu.VMEM((_N,), jnp.float32),
        pltpu.VMEM((_ROWS_PER_TILE * _L,), jnp.float32),
        pltpu.SemaphoreType.DMA,
    ],
)
def _sc_minima(p0_hbm, p1_hbm, p2_hbm, g0_hbm, g1_hbm, g2_hbm, rowph_hbm, colph_hbm, g0v, g1v, g2v, pv0, pv1, pv2, cminv, rowv, sem):
    _sc_body(p0_hbm, p1_hbm, p2_hbm, g0_hbm, g1_hbm, g2_hbm, rowph_hbm, colph_hbm, g0v, g1v, g2v, pv0, pv1, pv2, cminv, rowv, sem)


def _tc_merge(tcpart_ref, rowph_ref, colph_ref, out_ref):
    rmin = jnp.min(rowph_ref[...], axis=1, keepdims=True)
    fsum = jnp.sum(jnp.sqrt(rmin + _EPS), axis=(0, 1), keepdims=True)
    cmin = jnp.min(colph_ref[...], axis=0, keepdims=True)
    bsum = jnp.sum(jnp.sqrt(cmin + _EPS), axis=(0, 1), keepdims=True)
    ftot = tcpart_ref[:, 0:1] + fsum
    btot = tcpart_ref[:, 1:2] + bsum
    out_ref[...] = ftot / (_B * _M) + btot / (_B * _N)


def kernel(predict_pc, gt_pc):
    p_sc = predict_pc[_TCB]
    g_sc = gt_pc[_TCB]
    p_b = jnp.broadcast_to(p_sc[:, :, None], (_C, _M, _L)).reshape(_C, _M * _L)
    rowph, colph = _sc_minima(
        p_b[0], p_b[1], p_b[2], g_sc[0], g_sc[1], g_sc[2]
    )

    p_tc = predict_pc[:_TCB]
    g_tc = gt_pc[:_TCB]
    tc_part = pl.pallas_call(
        _tc_main,
        grid=(_TCB, _MI),
        in_specs=[
            pl.BlockSpec((1, _C, _MB), lambda b, mi: (b, 0, mi)),
            pl.BlockSpec((1, _C, _N), lambda b, mi: (b, 0, 0)),
        ],
        out_specs=pl.BlockSpec((1, 2), lambda b, mi: (0, 0)),
        out_shape=jax.ShapeDtypeStruct((1, 2), jnp.float32),
        scratch_shapes=[
            pltpu.VMEM((1, _N), jnp.float32),
            pltpu.VMEM((1, 1), jnp.float32),
            pltpu.VMEM((1, 1), jnp.float32),
        ],
    )(p_tc, g_tc)

    out = pl.pallas_call(
        _tc_merge,
        out_shape=jax.ShapeDtypeStruct((1, 1), jnp.float32),
    )(tc_part, rowph.reshape(_M, _L), colph.reshape(_NW, _N))
    return out[0, 0]


# speedup vs baseline: 1.3769x; 1.1549x over previous
import functools

import jax
import jax.numpy as jnp
from jax import lax
from jax.experimental import pallas as pl
from jax.experimental.pallas import tpu as pltpu
from jax.experimental.pallas import tpu_sc as plsc

_B, _C, _M = 4, 3, 4096
_N = 4096
_MB = 1024
_TCB = 3
_MI = _M // _MB
_EPS = 1e-8

_NC = 2
_NS = 16
_NW = _NC * _NS
_ROWS_PER_TILE = _M // _NW
_L = 16
_CHUNKS = _N // _L
_R = 8


def _tc_main(p_ref, g_ref, out_ref, colmin_ref, facc_ref, bacc_ref):
    b = pl.program_id(0)
    mi = pl.program_id(1)

    @pl.when(mi == 0)
    def _init_batch():
        colmin_ref[...] = jnp.full((1, _N), jnp.inf, jnp.float32)

    @pl.when((b == 0) & (mi == 0))
    def _init_all():
        facc_ref[...] = jnp.zeros((1, 1), jnp.float32)
        bacc_ref[...] = jnp.zeros((1, 1), jnp.float32)

    p = p_ref[0]
    g = g_ref[0]
    p0 = p[0, :].reshape(_MB, 1)
    p1 = p[1, :].reshape(_MB, 1)
    p2 = p[2, :].reshape(_MB, 1)
    g0 = g[0, :].reshape(1, _N)
    g1 = g[1, :].reshape(1, _N)
    g2 = g[2, :].reshape(1, _N)
    q0 = -2.0 * p0
    q1 = -2.0 * p1
    q2 = -2.0 * p2
    np2 = p0 * p0 + p1 * p1 + p2 * p2
    ng2 = g0 * g0 + g1 * g1 + g2 * g2
    t = q0 * g0 + q1 * g1 + q2 * g2
    e = t + ng2
    f = t + np2

    rowmin = np2 + jnp.min(e, axis=1, keepdims=True)
    facc_ref[...] += jnp.sum(jnp.sqrt(rowmin + _EPS), axis=(0, 1), keepdims=True)
    colmin_ref[...] = jnp.minimum(colmin_ref[...], jnp.min(f, axis=0, keepdims=True))

    @pl.when(mi == _MI - 1)
    def _finish_batch():
        bmin = ng2 + colmin_ref[...]
        bacc_ref[...] += jnp.sum(jnp.sqrt(bmin + _EPS), axis=(0, 1), keepdims=True)

    @pl.when((b == _TCB - 1) & (mi == _MI - 1))
    def _emit():
        out_ref[...] = jnp.concatenate([facc_ref[...], bacc_ref[...]], axis=1)


def _sc_body(p0_hbm, p1_hbm, p2_hbm, g0_hbm, g1_hbm, g2_hbm, rowph_hbm, colph_hbm, g0v, g1v, g2v, pv0, pv1, pv2, cminv, rowv, sem):
    cid = lax.axis_index("c")
    sid = lax.axis_index("s")
    wid = sid * _NC + cid
    base = wid * _ROWS_PER_TILE

    pltpu.sync_copy(g0_hbm, g0v)
    pltpu.sync_copy(g1_hbm, g1v)
    pltpu.sync_copy(g2_hbm, g2v)
    pltpu.sync_copy(p0_hbm.at[pl.ds(base * _L, _ROWS_PER_TILE * _L)], pv0)
    pltpu.sync_copy(p1_hbm.at[pl.ds(base * _L, _ROWS_PER_TILE * _L)], pv1)
    pltpu.sync_copy(p2_hbm.at[pl.ds(base * _L, _ROWS_PER_TILE * _L)], pv2)

    big = jnp.full((_L,), 3.0e38, jnp.float32)
    for j in range(_CHUNKS):
        cminv[pl.ds(j * _L, _L)] = big

    def block_body(mb, _):
        ob = mb * (_R * _L)
        ps = [
            (pv0[pl.ds(ob + r * _L, _L)],
             pv1[pl.ds(ob + r * _L, _L)],
             pv2[pl.ds(ob + r * _L, _L)])
            for r in range(_R)
        ]

        def chunk_body(j, raccs):
            o = j * _L
            g0c = g0v[pl.ds(o, _L)]
            g1c = g1v[pl.ds(o, _L)]
            g2c = g2v[pl.ds(o, _L)]
            ds_list = []
            for r in range(_R):
                d0 = ps[r][0] - g0c
                d1 = ps[r][1] - g1c
                d2 = ps[r][2] - g2c
                ds_list.append(d0 * d0 + d1 * d1 + d2 * d2)
            m = ds_list
            while len(m) > 1:
                m = [jnp.minimum(m[i], m[i + 1]) for i in range(0, len(m) - 1, 2)] + (
                    [m[-1]] if len(m) % 2 else []
                )
            cminv[pl.ds(o, _L)] = jnp.minimum(cminv[pl.ds(o, _L)], m[0])
            return tuple(jnp.minimum(raccs[r], ds_list[r]) for r in range(_R))

        raccs = lax.fori_loop(0, _CHUNKS, chunk_body, (big,) * _R, unroll=2)
        for r in range(_R):
            rowv[pl.ds(ob + r * _L, _L)] = raccs[r]
        return 0

    lax.fori_loop(0, _ROWS_PER_TILE // _R, block_body, 0)

    pltpu.sync_copy(rowv, rowph_hbm.at[pl.ds(base * _L, _ROWS_PER_TILE * _L)])
    pltpu.sync_copy(cminv, colph_hbm.at[pl.ds(wid * _N, _N)])


@functools.partial(
    pl.kernel,
    out_type=(
        jax.ShapeDtypeStruct((_M * _L,), jnp.float32),
        jax.ShapeDtypeStruct((_NW * _N,), jnp.float32),
    ),
    mesh=plsc.VectorSubcoreMesh(core_axis_name="c", subcore_axis_name="s"),
    scratch_types=[
        pltpu.VMEM((_N,), jnp.float32),
        pltpu.VMEM((_N,), jnp.float32),
        pltpu.VMEM((_N,), jnp.float32),
        pltpu.VMEM((_ROWS_PER_TILE * _L,), jnp.float32),
        pltpu.VMEM((_ROWS_PER_TILE * _L,), jnp.float32),
        pltpu.VMEM((_ROWS_PER_TILE * _L,), jnp.float32),
        pltpu.VMEM((_N,), jnp.float32),
        pltpu.VMEM((_ROWS_PER_TILE * _L,), jnp.float32),
        pltpu.SemaphoreType.DMA,
    ],
)
def _sc_minima(p0_hbm, p1_hbm, p2_hbm, g0_hbm, g1_hbm, g2_hbm, rowph_hbm, colph_hbm, g0v, g1v, g2v, pv0, pv1, pv2, cminv, rowv, sem):
    _sc_body(p0_hbm, p1_hbm, p2_hbm, g0_hbm, g1_hbm, g2_hbm, rowph_hbm, colph_hbm, g0v, g1v, g2v, pv0, pv1, pv2, cminv, rowv, sem)


def _tc_merge(tcpart_ref, rowph_ref, colph_ref, out_ref):
    rmin = jnp.min(rowph_ref[...], axis=1, keepdims=True)
    fsum = jnp.sum(jnp.sqrt(rmin + _EPS), axis=(0, 1), keepdims=True)
    cmin = jnp.min(colph_ref[...], axis=0, keepdims=True)
    bsum = jnp.sum(jnp.sqrt(cmin + _EPS), axis=(0, 1), keepdims=True)
    ftot = tcpart_ref[:, 0:1] + fsum
    btot = tcpart_ref[:, 1:2] + bsum
    out_ref[...] = ftot / (_B * _M) + btot / (_B * _N)


def kernel(predict_pc, gt_pc):
    p_sc = predict_pc[_TCB]
    g_sc = gt_pc[_TCB]
    p_b = jnp.broadcast_to(p_sc[:, :, None], (_C, _M, _L)).reshape(_C, _M * _L)
    rowph, colph = _sc_minima(
        p_b[0], p_b[1], p_b[2], g_sc[0], g_sc[1], g_sc[2]
    )

    p_tc = predict_pc[:_TCB]
    g_tc = gt_pc[:_TCB]
    tc_part = pl.pallas_call(
        _tc_main,
        grid=(_TCB, _MI),
        in_specs=[
            pl.BlockSpec((1, _C, _MB), lambda b, mi: (b, 0, mi)),
            pl.BlockSpec((1, _C, _N), lambda b, mi: (b, 0, 0)),
        ],
        out_specs=pl.BlockSpec((1, 2), lambda b, mi: (0, 0)),
        out_shape=jax.ShapeDtypeStruct((1, 2), jnp.float32),
        scratch_shapes=[
            pltpu.VMEM((1, _N), jnp.float32),
            pltpu.VMEM((1, 1), jnp.float32),
            pltpu.VMEM((1, 1), jnp.float32),
        ],
    )(p_tc, g_tc)

    out = pl.pallas_call(
        _tc_merge,
        out_shape=jax.ShapeDtypeStruct((1, 1), jnp.float32),
    )(tc_part, rowph.reshape(_M, _L), colph.reshape(_NW, _N))
    return out[0, 0]


# speedup vs baseline: 1.5816x; 1.1487x over previous
import functools

import jax
import jax.numpy as jnp
from jax import lax
from jax.experimental import pallas as pl
from jax.experimental.pallas import tpu as pltpu
from jax.experimental.pallas import tpu_sc as plsc

_B, _C, _M = 4, 3, 4096
_N = 4096
_MB = 1024
_TCB = 3
_MI = _M // _MB
_EPS = 1e-8

_NC = 2
_NS = 16
_NW = _NC * _NS
_RPT = _M // _NW
_L = 16
_CHUNKS = _N // _L
_R = 8
_GROUPS = _RPT // _L


def _tc_main(p_ref, g_ref, out_ref, colmin_ref, facc_ref, bacc_ref):
    b = pl.program_id(0)
    mi = pl.program_id(1)

    @pl.when(mi == 0)
    def _init_batch():
        colmin_ref[...] = jnp.full((1, _N), jnp.inf, jnp.float32)

    @pl.when((b == 0) & (mi == 0))
    def _init_all():
        facc_ref[...] = jnp.zeros((1, 1), jnp.float32)
        bacc_ref[...] = jnp.zeros((1, 1), jnp.float32)

    p = p_ref[0]
    g = g_ref[0]
    p0 = p[0, :].reshape(_MB, 1)
    p1 = p[1, :].reshape(_MB, 1)
    p2 = p[2, :].reshape(_MB, 1)
    g0 = g[0, :].reshape(1, _N)
    g1 = g[1, :].reshape(1, _N)
    g2 = g[2, :].reshape(1, _N)
    q0 = -2.0 * p0
    q1 = -2.0 * p1
    q2 = -2.0 * p2
    np2 = p0 * p0 + p1 * p1 + p2 * p2
    ng2 = g0 * g0 + g1 * g1 + g2 * g2
    t = q0 * g0 + q1 * g1 + q2 * g2
    e = t + ng2
    f = t + np2

    rowmin = np2 + jnp.min(e, axis=1, keepdims=True)
    facc_ref[...] += jnp.sum(jnp.sqrt(rowmin + _EPS), axis=(0, 1), keepdims=True)
    colmin_ref[...] = jnp.minimum(colmin_ref[...], jnp.min(f, axis=0, keepdims=True))

    @pl.when(mi == _MI - 1)
    def _finish_batch():
        bmin = ng2 + colmin_ref[...]
        bacc_ref[...] += jnp.sum(jnp.sqrt(bmin + _EPS), axis=(0, 1), keepdims=True)

    @pl.when((b == _TCB - 1) & (mi == _MI - 1))
    def _emit():
        out_ref[...] = jnp.concatenate([facc_ref[...], bacc_ref[...]], axis=1)


def _sc_body(pg_hbm, rowmin_hbm, colph_hbm, g0v, g1v, g2v, pc0, pc1, pc2, cminv, rowv, sem):
    cid = lax.axis_index("c")
    sid = lax.axis_index("s")
    wid = sid * _NC + cid
    base = wid * _RPT

    pltpu.sync_copy(pg_hbm.at[pl.ds(base, _RPT)], pc0)
    pltpu.sync_copy(pg_hbm.at[pl.ds(_M + base, _RPT)], pc1)
    pltpu.sync_copy(pg_hbm.at[pl.ds(2 * _M + base, _RPT)], pc2)
    pltpu.sync_copy(pg_hbm.at[pl.ds(3 * _M, _N)], g0v)
    pltpu.sync_copy(pg_hbm.at[pl.ds(3 * _M + _N, _N)], g1v)
    pltpu.sync_copy(pg_hbm.at[pl.ds(3 * _M + 2 * _N, _N)], g2v)

    big = jnp.full((_L,), 3.0e38, jnp.float32)
    for j in range(_CHUNKS):
        cminv[pl.ds(j * _L, _L)] = big

    def group_body(grp, _):
        og = grp * _L
        p0g = pc0[pl.ds(og, _L)]
        p1g = pc1[pl.ds(og, _L)]
        p2g = pc2[pl.ds(og, _L)]
        for half in range(2):
            ps = []
            for r in range(_R):
                l = half * _R + r
                ps.append((
                    jnp.broadcast_to(p0g[l], (_L,)),
                    jnp.broadcast_to(p1g[l], (_L,)),
                    jnp.broadcast_to(p2g[l], (_L,)),
                ))

            def chunk_body(j, raccs):
                o = j * _L
                g0c = g0v[pl.ds(o, _L)]
                g1c = g1v[pl.ds(o, _L)]
                g2c = g2v[pl.ds(o, _L)]
                ds_list = []
                for r in range(_R):
                    d0 = ps[r][0] - g0c
                    d1 = ps[r][1] - g1c
                    d2 = ps[r][2] - g2c
                    ds_list.append(d0 * d0 + d1 * d1 + d2 * d2)
                m = ds_list
                while len(m) > 1:
                    m = [jnp.minimum(m[i], m[i + 1]) for i in range(0, len(m) - 1, 2)] + (
                        [m[-1]] if len(m) % 2 else []
                    )
                cminv[pl.ds(o, _L)] = jnp.minimum(cminv[pl.ds(o, _L)], m[0])
                return tuple(jnp.minimum(raccs[r], ds_list[r]) for r in range(_R))

            raccs = lax.fori_loop(0, _CHUNKS, chunk_body, (big,) * _R, unroll=2)
            for r in range(_R):
                row = og + half * _R + r
                rowv[pl.ds(row * _L, _L)] = raccs[r]
        return 0

    lax.fori_loop(0, _GROUPS, group_body, 0)

    pltpu.sync_copy(rowv, rowmin_hbm.at[pl.ds(base * _L, _RPT * _L)])
    pltpu.sync_copy(cminv, colph_hbm.at[pl.ds(wid * _N, _N)])


@functools.partial(
    pl.kernel,
    out_type=(
        jax.ShapeDtypeStruct((_M * _L,), jnp.float32),
        jax.ShapeDtypeStruct((_NW * _N,), jnp.float32),
    ),
    mesh=plsc.VectorSubcoreMesh(core_axis_name="c", subcore_axis_name="s"),
    scratch_types=[
        pltpu.VMEM((_N,), jnp.float32),
        pltpu.VMEM((_N,), jnp.float32),
        pltpu.VMEM((_N,), jnp.float32),
        pltpu.VMEM((_RPT,), jnp.float32),
        pltpu.VMEM((_RPT,), jnp.float32),
        pltpu.VMEM((_RPT,), jnp.float32),
        pltpu.VMEM((_N,), jnp.float32),
        pltpu.VMEM((_RPT * _L,), jnp.float32),
        pltpu.SemaphoreType.DMA,
    ],
)
def _sc_minima(pg_hbm, rowmin_hbm, colph_hbm, g0v, g1v, g2v, pc0, pc1, pc2, cminv, rowv, sem):
    _sc_body(pg_hbm, rowmin_hbm, colph_hbm, g0v, g1v, g2v, pc0, pc1, pc2, cminv, rowv, sem)


def _tc_merge(tcpart_ref, rowmin_ref, colph_ref, out_ref):
    rmin = jnp.min(rowmin_ref[...], axis=1, keepdims=True)
    fsum = jnp.sum(jnp.sqrt(rmin + _EPS), axis=(0, 1), keepdims=True)
    cmin = jnp.min(colph_ref[...], axis=0)
    bsum = jnp.sum(jnp.sqrt(cmin + _EPS), axis=(0, 1)).reshape(1, 1)
    ftot = tcpart_ref[:, 0:1] + fsum
    btot = tcpart_ref[:, 1:2] + bsum
    out_ref[...] = ftot / (_B * _M) + btot / (_B * _N)


def kernel(predict_pc, gt_pc):
    pg = jnp.concatenate(
        [predict_pc[_TCB].reshape(-1), gt_pc[_TCB].reshape(-1)]
    )
    rowmin_sc, colph = _sc_minima(pg)

    tc_part = pl.pallas_call(
        _tc_main,
        grid=(_TCB, _MI),
        in_specs=[
            pl.BlockSpec((1, _C, _MB), lambda b, mi: (b, 0, mi)),
            pl.BlockSpec((1, _C, _N), lambda b, mi: (b, 0, 0)),
        ],
        out_specs=pl.BlockSpec((1, 2), lambda b, mi: (0, 0)),
        out_shape=jax.ShapeDtypeStruct((1, 2), jnp.float32),
        scratch_shapes=[
            pltpu.VMEM((1, _N), jnp.float32),
            pltpu.VMEM((1, 1), jnp.float32),
            pltpu.VMEM((1, 1), jnp.float32),
        ],
    )(predict_pc, gt_pc)

    out = pl.pallas_call(
        _tc_merge,
        out_shape=jax.ShapeDtypeStruct((1, 1), jnp.float32),
    )(tc_part, rowmin_sc.reshape(_M, _L), colph.reshape(_NW, 32, 128))
    return out[0, 0]
